# baseline (speedup 1.0000x reference)
import jax
import jax.numpy as jnp
from jax.experimental import pallas as pl
from jax.experimental.pallas import tpu as pltpu


def _round_up(v, m):
    return ((v + m - 1) // m) * m


def _build_wT(b_sm, a_ref, w_ref, K, N):
    sk = (jax.lax.shift_right_logical(
              jax.lax.broadcasted_iota(jnp.int32, (K, K // 4), 0), 2)
          == jax.lax.broadcasted_iota(jnp.int32, (K, K // 4), 1))
    rn = (jax.lax.shift_right_logical(
              jax.lax.broadcasted_iota(jnp.int32, (N // 4, N), 1), 2)
          == jax.lax.broadcasted_iota(jnp.int32, (N // 4, N), 0))
    at_rn = jax.lax.dot_general(
        a_ref[...], rn.astype(jnp.float32),
        (((0,), (0,)), ((), ())), preferred_element_type=jnp.float32)
    arep = jax.lax.dot_general(
        sk.astype(jnp.float32), at_rn,
        (((1,), (0,)), ((), ())), preferred_element_type=jnp.float32)

    lane = jax.lax.broadcasted_iota(jnp.int32, (8, N), 1) & 3
    krow = jax.lax.broadcasted_iota(jnp.int32, (8, N), 0) & 3
    t8 = jnp.zeros((8, N), jnp.float32)
    for p in range(4):
        for q in range(4):
            t8 = jnp.where((lane == p) & (krow == q), b_sm[p, q], t8)
    tpat = jnp.tile(t8, (K // 8, 1))
    w_ref[...] = arep * tpat


def _fused_body(b_sm, x_hbm, a_ref, bias_ref, o_hbm, xbuf, obuf, w_ref,
                in_sem, out_sem, *, TB, n_steps, rows_per_core):
    K, N = w_ref.shape
    core = pl.program_id(0)
    base = core * rows_per_core

    def dma_in(slot, step):
        pltpu.make_async_copy(
            x_hbm.at[pl.ds(base + step * TB, TB), :],
            xbuf.at[slot], in_sem.at[slot]).start()

    def wait_in(slot):
        pltpu.make_async_copy(
            xbuf.at[slot], xbuf.at[slot], in_sem.at[slot]).wait()

    def dma_out(slot, step):
        pltpu.make_async_copy(
            obuf.at[slot],
            o_hbm.at[pl.ds(base + step * TB, TB), :], out_sem.at[slot]).start()

    def wait_out(slot):
        pltpu.make_async_copy(
            obuf.at[slot], obuf.at[slot], out_sem.at[slot]).wait()

    dma_in(0, 0)
    _build_wT(b_sm, a_ref, w_ref, K, N)

    for step in range(n_steps):
        cur = step % 2
        if step + 1 < n_steps:
            dma_in((step + 1) % 2, step + 1)
        wait_in(cur)
        if step >= 2:
            wait_out(cur)
        acc = jnp.dot(xbuf[cur], w_ref[...],
                      preferred_element_type=jnp.float32)
        obuf[cur] = acc + bias_ref[...]
        dma_out(cur, step)
    for step in range(max(0, n_steps - 2), n_steps):
        wait_out(step % 2)


def kernel(x, A, B, bias):
    import functools

    M, K = x.shape
    A_N, A_K = A.shape
    B_N, B_K = B.shape
    N = A_N * B_N

    if bias is None:
        bias_row = jnp.zeros((1, N), jnp.float32)
    else:
        bias_row = bias.astype(jnp.float32).reshape(1, N)

    TB = min(4096, _round_up(M, 8))
    Mp = _round_up(M, 2 * TB)
    x_p = x if Mp == M else jnp.pad(x, ((0, Mp - M), (0, 0)))
    rows_per_core = Mp // 2
    n_steps = rows_per_core // TB

    out = pl.pallas_call(
        functools.partial(_fused_body, TB=TB, n_steps=n_steps,
                          rows_per_core=rows_per_core),
        out_shape=jax.ShapeDtypeStruct((Mp, N), jnp.float32),
        grid=(2,),
        in_specs=[
            pl.BlockSpec(memory_space=pltpu.MemorySpace.SMEM),
            pl.BlockSpec(memory_space=pltpu.MemorySpace.HBM),
            pl.BlockSpec((A_N, A_K), lambda c: (0, 0)),
            pl.BlockSpec((1, N), lambda c: (0, 0)),
        ],
        out_specs=pl.BlockSpec(memory_space=pltpu.MemorySpace.HBM),
        scratch_shapes=[
            pltpu.VMEM((min(2, n_steps), TB, K), jnp.float32),
            pltpu.VMEM((min(2, n_steps), TB, N), jnp.float32),
            pltpu.VMEM((K, N), jnp.float32),
            pltpu.SemaphoreType.DMA((2,)),
            pltpu.SemaphoreType.DMA((2,)),
        ],
        compiler_params=pltpu.CompilerParams(
            dimension_semantics=("parallel",),
            vmem_limit_bytes=50 * 1024 * 1024,
        ),
    )(B.astype(jnp.float32), x_p, A, bias_row)
    if Mp != M:
        out = out[:M]
    return out


# speedup vs baseline: 1.3365x; 1.3365x over previous
import jax
import jax.numpy as jnp
from jax.experimental import pallas as pl
from jax.experimental.pallas import tpu as pltpu


def _round_up(v, m):
    return ((v + m - 1) // m) * m


def _build_wT(b_sm, a_ref, w_ref, K, N):
    sk = (jax.lax.shift_right_logical(
              jax.lax.broadcasted_iota(jnp.int32, (K, K // 4), 0), 2)
          == jax.lax.broadcasted_iota(jnp.int32, (K, K // 4), 1))
    rn = (jax.lax.shift_right_logical(
              jax.lax.broadcasted_iota(jnp.int32, (N // 4, N), 1), 2)
          == jax.lax.broadcasted_iota(jnp.int32, (N // 4, N), 0))
    at_rn = jax.lax.dot_general(
        a_ref[...], rn.astype(jnp.float32),
        (((0,), (0,)), ((), ())), preferred_element_type=jnp.float32)
    arep = jax.lax.dot_general(
        sk.astype(jnp.float32), at_rn,
        (((1,), (0,)), ((), ())), preferred_element_type=jnp.float32)

    lane = jax.lax.broadcasted_iota(jnp.int32, (1, N), 1) & 3
    rows = []
    for q in range(4):
        rv = jnp.zeros((1, N), jnp.float32)
        for p in range(4):
            rv = jnp.where(lane == p, b_sm[p, q], rv)
        rows.append(rv)
    krow = jax.lax.broadcasted_iota(jnp.int32, (K, 1), 0) & 3
    tpat = jnp.where(krow == 0, rows[0],
                     jnp.where(krow == 1, rows[1],
                               jnp.where(krow == 2, rows[2], rows[3])))
    w_ref[...] = arep * tpat


def _fused_body(b_sm, x_ref, a_ref, bias_ref, o_ref, w_ref):
    K, N = w_ref.shape

    @pl.when(pl.program_id(1) == 0)
    def _():
        _build_wT(b_sm, a_ref, w_ref, K, N)

    acc = jnp.dot(x_ref[...], w_ref[...], preferred_element_type=jnp.float32)
    o_ref[...] = acc + bias_ref[...]


def kernel(x, A, B, bias):
    M, K = x.shape
    A_N, A_K = A.shape
    B_N, B_K = B.shape
    N = A_N * B_N

    if bias is None:
        bias_row = jnp.zeros((1, N), jnp.float32)
    else:
        bias_row = bias.astype(jnp.float32).reshape(1, N)

    TM = min(2048, _round_up(M, 8))
    Mp = _round_up(M, 2 * TM)
    x_p = x if Mp == M else jnp.pad(x, ((0, Mp - M), (0, 0)))
    J = Mp // TM // 2

    out = pl.pallas_call(
        _fused_body,
        out_shape=jax.ShapeDtypeStruct((Mp, N), jnp.float32),
        grid=(2, J),
        in_specs=[
            pl.BlockSpec(memory_space=pltpu.MemorySpace.SMEM),
            pl.BlockSpec((TM, K), lambda c, j: (c * J + j, 0)),
            pl.BlockSpec((A_N, A_K), lambda c, j: (0, 0)),
            pl.BlockSpec((1, N), lambda c, j: (0, 0)),
        ],
        out_specs=pl.BlockSpec((TM, N), lambda c, j: (c * J + j, 0)),
        scratch_shapes=[pltpu.VMEM((K, N), jnp.float32)],
        compiler_params=pltpu.CompilerParams(
            dimension_semantics=("parallel", "arbitrary"),
            vmem_limit_bytes=50 * 1024 * 1024,
        ),
    )(B.astype(jnp.float32), x_p, A, bias_row)
    if Mp != M:
        out = out[:M]
    return out


# speedup vs baseline: 1.3604x; 1.0179x over previous
import jax
import jax.numpy as jnp
from jax.experimental import pallas as pl
from jax.experimental.pallas import tpu as pltpu


def _round_up(v, m):
    return ((v + m - 1) // m) * m


def _build_wT(b_sm, a_ref, w_ref, K, N):
    sk = (jax.lax.shift_right_logical(
              jax.lax.broadcasted_iota(jnp.int32, (K, K // 4), 0), 2)
          == jax.lax.broadcasted_iota(jnp.int32, (K, K // 4), 1))
    rn = (jax.lax.shift_right_logical(
              jax.lax.broadcasted_iota(jnp.int32, (N // 4, N), 1), 2)
          == jax.lax.broadcasted_iota(jnp.int32, (N // 4, N), 0))
    at_rn = jax.lax.dot_general(
        a_ref[...], rn.astype(jnp.float32),
        (((0,), (0,)), ((), ())), preferred_element_type=jnp.float32)
    arep = jax.lax.dot_general(
        sk.astype(jnp.float32), at_rn,
        (((1,), (0,)), ((), ())), preferred_element_type=jnp.float32)

    lane = jax.lax.broadcasted_iota(jnp.int32, (8, N), 1) & 3
    krow = jax.lax.broadcasted_iota(jnp.int32, (8, N), 0) & 3
    t8 = jnp.zeros((8, N), jnp.float32)
    for p in range(4):
        for q in range(4):
            t8 = jnp.where((lane == p) & (krow == q), b_sm[p, q], t8)
    tpat = jnp.tile(t8, (K // 8, 1))
    w_ref[...] = arep * tpat


def _fused_body(b_sm, x_ref, a_ref, bias_ref, o_ref, w_ref):
    K, N = w_ref.shape

    @pl.when(pl.program_id(1) == 0)
    def _():
        _build_wT(b_sm, a_ref, w_ref, K, N)

    acc = jnp.dot(x_ref[...], w_ref[...], preferred_element_type=jnp.float32)
    o_ref[...] = acc + bias_ref[...]


def kernel(x, A, B, bias):
    M, K = x.shape
    A_N, A_K = A.shape
    B_N, B_K = B.shape
    N = A_N * B_N

    if bias is None:
        bias_row = jnp.zeros((1, N), jnp.float32)
    else:
        bias_row = bias.astype(jnp.float32).reshape(1, N)

    TM = min(2048, _round_up(M, 8))
    Mp = _round_up(M, 2 * TM)
    x_p = x if Mp == M else jnp.pad(x, ((0, Mp - M), (0, 0)))
    J = Mp // TM // 2

    out = pl.pallas_call(
        _fused_body,
        out_shape=jax.ShapeDtypeStruct((Mp, N), jnp.float32),
        grid=(2, J),
        in_specs=[
            pl.BlockSpec(memory_space=pltpu.MemorySpace.SMEM),
            pl.BlockSpec((TM, K), lambda c, j: (c * J + j, 0)),
            pl.BlockSpec((A_N, A_K), lambda c, j: (0, 0)),
            pl.BlockSpec((1, N), lambda c, j: (0, 0)),
        ],
        out_specs=pl.BlockSpec((TM, N), lambda c, j: (c * J + j, 0)),
        scratch_shapes=[pltpu.VMEM((K, N), jnp.float32)],
        compiler_params=pltpu.CompilerParams(
            dimension_semantics=("parallel", "arbitrary"),
            vmem_limit_bytes=50 * 1024 * 1024,
        ),
    )(B.astype(jnp.float32), x_p, A, bias_row)
    if Mp != M:
        out = out[:M]
    return out
